# Initial kernel scaffold; baseline (speedup 1.0000x reference)
import jax
import jax.numpy as jnp
from jax.experimental import pallas as pl


def kernel(x, edge_index, rm_w1, rm_b1, rm_w2, rm_b2, rm_w3, rm_b3, rm_w4, rm_b4, om_w1, om_b1, om_w2, om_b2):
    raise NotImplementedError("write your pallas kernel here")


# speedup vs baseline: 2.2409x; 2.2409x over previous
import functools

import jax
import jax.numpy as jnp
from jax import lax
from jax.experimental import pallas as pl
from jax.experimental.pallas import tpu as pltpu
from jax.experimental.pallas import tpu_sc as plsc

N = 50000
E = 800000
EP = 802816
ROWS = EP // 128
HID = 100

NW = 32
W_ROWS = ROWS // NW
GK = 14
GCH = GK * 128
G_NCHUNK = W_ROWS // GK

TROWS = ROWS // 16
SK = 4
SCH = SK * 128
S_NCHUNK = TROWS // SK
NPAD = 50008
NT = N // 16
Q = 32
HIDP = 128

_sc_mesh = plsc.VectorSubcoreMesh(core_axis_name="c", subcore_axis_name="s")


def _gather_body(x8, sg, rg, spos, rpos, sidx, ridx, srows, rrows, sem):
    c = lax.axis_index("c")
    s = lax.axis_index("s")
    wid = s * 2 + c
    row_base = wid * W_ROWS

    def chunk(i, carry):
        r0 = row_base + i * GK
        pltpu.sync_copy(sg.at[pl.ds(r0, GK)], sidx)
        pltpu.sync_copy(rg.at[pl.ds(r0, GK)], ridx)
        cps = []
        for j in range(GK):
            cps.append(pltpu.async_copy(
                x8.at[sidx.at[j]], srows.at[pl.ds(j * 128, 128)], sem))
            cps.append(pltpu.async_copy(
                x8.at[ridx.at[j]], rrows.at[pl.ds(j * 128, 128)], sem))
        for cp in cps:
            cp.wait()
        e0 = r0 * 128
        pltpu.sync_copy(srows, spos.at[pl.ds(e0, GCH)])
        pltpu.sync_copy(rrows, rpos.at[pl.ds(e0, GCH)])
        return carry

    lax.fori_loop(0, G_NCHUNK, chunk, 0)


def _scatter_body(e0, e1, e2, e3, ss, zeros, agg0, agg1, agg2, agg3,
                  acc, sidx, ebuf, sem):
    c = lax.axis_index("c")
    s = lax.axis_index("s")

    def do_quarter(eq, aggq):
        pltpu.sync_copy(zeros, acc.at[pl.ds(s * NT, NT)])
        plsc.subcore_barrier()

        def chunk(i, carry):
            r0 = s * TROWS + i * SK
            eoff = r0 * 128
            pltpu.sync_copy(ss.at[pl.ds(r0, SK)], sidx)
            pltpu.sync_copy(eq.at[pl.ds(eoff, SCH)], ebuf)
            cps = [
                pltpu.async_copy(
                    ebuf.at[pl.ds(j * 128, 128)], acc.at[sidx.at[j]],
                    sem, add=True)
                for j in range(SK)
            ]
            for cp in cps:
                cp.wait()
            return carry

        lax.fori_loop(0, S_NCHUNK, chunk, 0)
        plsc.subcore_barrier()
        pltpu.sync_copy(acc.at[pl.ds(s * NT, NT)], aggq.at[pl.ds(s * NT, NT)])
        plsc.subcore_barrier()

    @pl.when(c == 0)
    def _():
        do_quarter(e0, agg0)
        do_quarter(e2, agg2)

    @pl.when(c == 1)
    def _():
        do_quarter(e1, agg1)
        do_quarter(e3, agg3)


def _relation_mlp_body(sp, rp, w1s, w1r, b1, w2, b2, w3, b3, w4, b4,
                       out0, out1, out2, out3):
    f32 = jnp.float32
    h = jnp.dot(sp[...], w1s[...], preferred_element_type=f32)
    h = h + jnp.dot(rp[...], w1r[...], preferred_element_type=f32) + b1[...]
    h = jnp.maximum(h, 0.0)
    h = jnp.maximum(jnp.dot(h, w2[...], preferred_element_type=f32) + b2[...], 0.0)
    h = jnp.maximum(jnp.dot(h, w3[...], preferred_element_type=f32) + b3[...], 0.0)
    h = jnp.maximum(jnp.dot(h, w4[...], preferred_element_type=f32) + b4[...], 0.0)
    out0[...] = h[:, 0 * Q:1 * Q]
    out1[...] = h[:, 1 * Q:2 * Q]
    out2[...] = h[:, 2 * Q:3 * Q]
    out3[...] = h[:, 3 * Q:4 * Q]


def _object_mlp_body(x, a0, a1, a2, a3, w1x, w1a, b1, w2, b2, out):
    f32 = jnp.float32
    agg = jnp.concatenate([a0[...], a1[...], a2[...], a3[...]], axis=1)
    h = jnp.dot(x[...], w1x[...], preferred_element_type=f32)
    h = h + jnp.dot(agg, w1a[...], preferred_element_type=f32) + b1[...]
    h = jnp.maximum(h, 0.0)
    out[...] = jnp.dot(h, w2[...], preferred_element_type=f32) + b2[...]


ET = 4096
E_GRID = EP // ET
NTC = 2000
N_GRID = N // NTC


def _full(shape):
    return pl.BlockSpec(shape, lambda i: (0,) * len(shape))


def _run_gather(x8, sg, rg):
    i32 = jnp.int32
    f32 = jnp.float32
    return pl.kernel(
        _gather_body,
        out_type=(jax.ShapeDtypeStruct((EP, 8), f32),
                  jax.ShapeDtypeStruct((EP, 8), f32)),
        mesh=_sc_mesh,
        compiler_params=pltpu.CompilerParams(use_tc_tiling_on_sc=False),
        scratch_types=[
            pltpu.VMEM((GK, 128), i32),
            pltpu.VMEM((GK, 128), i32),
            pltpu.VMEM((GCH, 8), f32),
            pltpu.VMEM((GCH, 8), f32),
            pltpu.SemaphoreType.DMA,
        ],
    )(x8, sg, rg)


def _run_relation_mlp(spos, rpos, rm_w1, rm_b1, rm_w2, rm_b2, rm_w3, rm_b3,
                      rm_w4, rm_b4):
    f32 = jnp.float32
    w1s = jnp.pad(rm_w1[0:2], ((0, 6), (0, 0)))
    w1r = jnp.pad(rm_w1[2:4], ((0, 6), (0, 0)))
    w4p = jnp.pad(rm_w4, ((0, 0), (0, HIDP - HID)))
    b4p = jnp.pad(rm_b4, (0, HIDP - HID))
    return pl.pallas_call(
        _relation_mlp_body,
        grid=(E_GRID,),
        in_specs=[
            pl.BlockSpec((ET, 8), lambda i: (i, 0)),
            pl.BlockSpec((ET, 8), lambda i: (i, 0)),
            _full((8, HID)), _full((8, HID)), _full((1, HID)),
            _full((HID, HID)), _full((1, HID)),
            _full((HID, HID)), _full((1, HID)),
            _full((HID, HIDP)), _full((1, HIDP)),
        ],
        out_specs=[pl.BlockSpec((ET, Q), lambda i: (i, 0))] * 4,
        out_shape=[jax.ShapeDtypeStruct((EP, Q), f32)] * 4,
    )(spos, rpos, w1s, w1r, rm_b1.reshape(1, HID),
      rm_w2, rm_b2.reshape(1, HID), rm_w3, rm_b3.reshape(1, HID),
      w4p, b4p.reshape(1, HIDP))


def _run_scatter(equarters, ss, zeros):
    i32 = jnp.int32
    f32 = jnp.float32
    return pl.kernel(
        _scatter_body,
        out_type=tuple(jax.ShapeDtypeStruct((N, Q), f32) for _ in range(4)),
        mesh=_sc_mesh,
        compiler_params=pltpu.CompilerParams(use_tc_tiling_on_sc=False),
        scratch_types=[
            pltpu.VMEM_SHARED((NPAD, Q), f32),
            pltpu.VMEM((SK, 128), i32),
            pltpu.VMEM((SCH, Q), f32),
            pltpu.SemaphoreType.DMA,
        ],
    )(*equarters, ss, zeros)


def _run_object_mlp(x, aggq, om_w1, om_b1, om_w2, om_b2):
    f32 = jnp.float32
    return pl.pallas_call(
        _object_mlp_body,
        grid=(N_GRID,),
        in_specs=[
            pl.BlockSpec((NTC, 2), lambda i: (i, 0)),
            pl.BlockSpec((NTC, Q), lambda i: (i, 0)),
            pl.BlockSpec((NTC, Q), lambda i: (i, 0)),
            pl.BlockSpec((NTC, Q), lambda i: (i, 0)),
            pl.BlockSpec((NTC, Q), lambda i: (i, 0)),
            _full((2, HID)), _full((HIDP, HID)), _full((1, HID)),
            _full((HID, 2)), _full((1, 2)),
        ],
        out_specs=pl.BlockSpec((NTC, 2), lambda i: (i, 0)),
        out_shape=jax.ShapeDtypeStruct((N, 2), f32),
    )(x, *aggq, om_w1[0:2], jnp.pad(om_w1[2:], ((0, HIDP - HID), (0, 0))),
      om_b1.reshape(1, HID),
      om_w2, om_b2.reshape(1, 2))


def _prep_indices(x, edge_index):
    sender = edge_index[0]
    receiver = edge_index[1]
    pad = EP - E
    sg = jnp.pad(sender, (0, pad)).reshape(ROWS, 128)
    rg = jnp.pad(receiver, (0, pad)).reshape(ROWS, 128)
    ss = jnp.pad(sender, (0, pad), constant_values=N).reshape(ROWS, 128)
    x8 = jnp.pad(x, ((0, 0), (0, 6)))
    zeros = jnp.zeros((NT, Q), jnp.float32)
    return x8, sg, rg, ss, zeros


def kernel(x, edge_index, rm_w1, rm_b1, rm_w2, rm_b2, rm_w3, rm_b3,
           rm_w4, rm_b4, om_w1, om_b1, om_w2, om_b2):
    x8, sg, rg, ss, zeros = _prep_indices(x, edge_index)
    spos, rpos = _run_gather(x8, sg, rg)
    equarters = _run_relation_mlp(spos, rpos, rm_w1, rm_b1, rm_w2, rm_b2,
                                  rm_w3, rm_b3, rm_w4, rm_b4)
    aggq = _run_scatter(equarters, ss, zeros)
    return _run_object_mlp(x, aggq, om_w1, om_b1, om_w2, om_b2)


# speedup vs baseline: 4.5961x; 2.0510x over previous
import jax
import jax.numpy as jnp
from jax import lax
from jax.experimental import pallas as pl
from jax.experimental.pallas import tpu as pltpu
from jax.experimental.pallas import tpu_sc as plsc

N = 50000
E = 800000
EP = 802816
ROWS = EP // 128
HID = 100
HIDP = 128

NW = 32
W_ROWS = ROWS // NW
GK = 14
GCH = GK * 128
G_NCHUNK = W_ROWS // GK

TROWS = ROWS // 16
SK = 2
SCH = SK * 128
S_NCHUNK = TROWS // SK
NPAD = 50008
NT = N // 16
Q = 32

_sc_mesh = plsc.VectorSubcoreMesh(core_axis_name="c", subcore_axis_name="s")
_sc_params = pltpu.CompilerParams(use_tc_tiling_on_sc=False)


def _gather_body(x8, sg, rg, spos, rpos,
                 sidx0, ridx0, srows0, rrows0,
                 sidx1, ridx1, srows1, rrows1,
                 isem, gsem, wsem):
    c = lax.axis_index("c")
    s = lax.axis_index("s")
    wid = s * 2 + c
    row_base = wid * W_ROWS
    bufs = ((sidx0, ridx0, srows0, rrows0), (sidx1, ridx1, srows1, rrows1))

    def load_idx(i, b):
        sidx, ridx, _, _ = bufs[b]
        r0 = row_base + i * GK
        pltpu.async_copy(sg.at[pl.ds(r0, GK)], sidx, isem)
        pltpu.async_copy(rg.at[pl.ds(r0, GK)], ridx, isem)

    def drain_idx(b):
        sidx, ridx, _, _ = bufs[b]
        pltpu.make_async_copy(sg.at[pl.ds(0, GK)], sidx, isem).wait()
        pltpu.make_async_copy(rg.at[pl.ds(0, GK)], ridx, isem).wait()

    def run_chunk(i, b):
        sidx, ridx, srows, rrows = bufs[b]
        cps = []
        for j in range(GK):
            cps.append(pltpu.async_copy(
                x8.at[sidx.at[j]], srows.at[pl.ds(j * 128, 128)], gsem))
            cps.append(pltpu.async_copy(
                x8.at[ridx.at[j]], rrows.at[pl.ds(j * 128, 128)], gsem))
        for cp in cps:
            cp.wait()
        e0 = (row_base + i * GK) * 128
        pltpu.async_copy(srows, spos.at[pl.ds(e0, GCH)], wsem)
        pltpu.async_copy(rrows, rpos.at[pl.ds(e0, GCH)], wsem)

    def drain_wb(b):
        _, _, srows, rrows = bufs[b]
        pltpu.make_async_copy(srows, spos.at[pl.ds(0, GCH)], wsem).wait()
        pltpu.make_async_copy(rrows, rpos.at[pl.ds(0, GCH)], wsem).wait()

    load_idx(0, 0)

    def pair(p, carry):
        i = p * 2
        drain_idx(0)
        load_idx(i + 1, 1)

        @pl.when(p > 0)
        def _():
            drain_wb(0)
        run_chunk(i, 0)
        drain_idx(1)

        @pl.when(p > 0)
        def _():
            drain_wb(1)
        run_chunk(i + 1, 1)

        @pl.when(p + 1 < G_NCHUNK // 2)
        def _():
            load_idx(i + 2, 0)
        return carry

    lax.fori_loop(0, G_NCHUNK // 2, pair, 0)
    drain_wb(0)
    drain_wb(1)


def _scatter_body(e, ss, zeros, agg, acc,
                  sidx0, ebuf0, sidx1, ebuf1, lsem, csem):
    c = lax.axis_index("c")
    s = lax.axis_index("s")
    bufs = ((sidx0, ebuf0), (sidx1, ebuf1))

    def do_quarter(qoff):
        pltpu.sync_copy(zeros, acc.at[pl.ds(s * NT, NT)])
        plsc.subcore_barrier()

        def load(i, b):
            sidx, ebuf = bufs[b]
            r0 = s * TROWS + i * SK
            pltpu.async_copy(ss.at[pl.ds(r0, SK)], sidx, lsem)
            pltpu.async_copy(
                e.at[pl.ds(r0 * 128, SCH), pl.ds(qoff, Q)], ebuf, lsem)

        def drain_load(b):
            sidx, ebuf = bufs[b]
            pltpu.make_async_copy(ss.at[pl.ds(0, SK)], sidx, lsem).wait()
            pltpu.make_async_copy(
                e.at[pl.ds(0, SCH), pl.ds(qoff, Q)], ebuf, lsem).wait()

        def fire_scatter(b):
            sidx, ebuf = bufs[b]
            for j in range(SK):
                pltpu.async_copy(ebuf.at[pl.ds(j * 128, 128)],
                                 acc.at[sidx.at[j]], csem, add=True)

        def drain_scatter(b):
            sidx, ebuf = bufs[b]
            for j in range(SK):
                pltpu.make_async_copy(ebuf.at[pl.ds(j * 128, 128)],
                                      acc.at[sidx.at[0]], csem).wait()

        load(0, 0)

        def pair(p, carry):
            i = p * 2

            @pl.when(p > 0)
            def _():
                drain_scatter(1)
            load(i + 1, 1)
            drain_load(0)
            fire_scatter(0)
            drain_load(1)
            drain_scatter(0)

            @pl.when(p + 1 < S_NCHUNK // 2)
            def _():
                load(i + 2, 0)
            fire_scatter(1)
            return carry

        lax.fori_loop(0, S_NCHUNK // 2, pair, 0)
        drain_scatter(1)
        plsc.subcore_barrier()
        pltpu.sync_copy(acc.at[pl.ds(s * NT, NT)],
                        agg.at[pl.ds(s * NT, NT), pl.ds(qoff, Q)])
        plsc.subcore_barrier()

    @pl.when(c == 0)
    def _():
        do_quarter(0)
        do_quarter(2 * Q)

    @pl.when(c == 1)
    def _():
        do_quarter(Q)
        do_quarter(3 * Q)


def _relation_mlp_body(sp, rp, w1s, w1r, b1, w2, b2, w3, b3, w4, b4, out):
    f32 = jnp.float32
    h = jnp.dot(sp[...], w1s[...], preferred_element_type=f32)
    h = h + jnp.dot(rp[...], w1r[...], preferred_element_type=f32) + b1[...]
    h = jnp.maximum(h, 0.0)
    h = jnp.maximum(jnp.dot(h, w2[...], preferred_element_type=f32) + b2[...], 0.0)
    h = jnp.maximum(jnp.dot(h, w3[...], preferred_element_type=f32) + b3[...], 0.0)
    h = jnp.maximum(jnp.dot(h, w4[...], preferred_element_type=f32) + b4[...], 0.0)
    out[...] = h


def _object_mlp_body(x, agg, w1x, w1a, b1, w2, b2, out):
    f32 = jnp.float32
    h = jnp.dot(x[...], w1x[...], preferred_element_type=f32)
    h = h + jnp.dot(agg[...], w1a[...], preferred_element_type=f32) + b1[...]
    h = jnp.maximum(h, 0.0)
    out[...] = jnp.dot(h, w2[...], preferred_element_type=f32) + b2[...]


ET = 4096
E_GRID = EP // ET
NTC = 2000
N_GRID = N // NTC


def _full(shape):
    return pl.BlockSpec(shape, lambda i: (0,) * len(shape))


def _run_gather(x8, sg, rg):
    i32 = jnp.int32
    f32 = jnp.float32
    return pl.kernel(
        _gather_body,
        out_type=(jax.ShapeDtypeStruct((EP, 8), f32),
                  jax.ShapeDtypeStruct((EP, 8), f32)),
        mesh=_sc_mesh,
        compiler_params=_sc_params,
        scratch_types=[
            pltpu.VMEM((GK, 128), i32), pltpu.VMEM((GK, 128), i32),
            pltpu.VMEM((GCH, 8), f32), pltpu.VMEM((GCH, 8), f32),
            pltpu.VMEM((GK, 128), i32), pltpu.VMEM((GK, 128), i32),
            pltpu.VMEM((GCH, 8), f32), pltpu.VMEM((GCH, 8), f32),
            pltpu.SemaphoreType.DMA,
            pltpu.SemaphoreType.DMA,
            pltpu.SemaphoreType.DMA,
        ],
    )(x8, sg, rg)


def _run_relation_mlp(spos, rpos, rm_w1, rm_b1, rm_w2, rm_b2, rm_w3, rm_b3,
                      rm_w4, rm_b4):
    f32 = jnp.float32
    w1s = jnp.pad(rm_w1[0:2], ((0, 6), (0, 0)))
    w1r = jnp.pad(rm_w1[2:4], ((0, 6), (0, 0)))
    w4p = jnp.pad(rm_w4, ((0, 0), (0, HIDP - HID)))
    b4p = jnp.pad(rm_b4, (0, HIDP - HID))
    return pl.pallas_call(
        _relation_mlp_body,
        grid=(E_GRID,),
        in_specs=[
            pl.BlockSpec((ET, 8), lambda i: (i, 0)),
            pl.BlockSpec((ET, 8), lambda i: (i, 0)),
            _full((8, HID)), _full((8, HID)), _full((1, HID)),
            _full((HID, HID)), _full((1, HID)),
            _full((HID, HID)), _full((1, HID)),
            _full((HID, HIDP)), _full((1, HIDP)),
        ],
        out_specs=pl.BlockSpec((ET, HIDP), lambda i: (i, 0)),
        out_shape=jax.ShapeDtypeStruct((EP, HIDP), f32),
        compiler_params=pltpu.CompilerParams(
            dimension_semantics=("arbitrary",)),
    )(spos, rpos, w1s, w1r, rm_b1.reshape(1, HID),
      rm_w2, rm_b2.reshape(1, HID), rm_w3, rm_b3.reshape(1, HID),
      w4p, b4p.reshape(1, HIDP))


def _run_scatter(e, ss, zeros):
    i32 = jnp.int32
    f32 = jnp.float32
    return pl.kernel(
        _scatter_body,
        out_type=jax.ShapeDtypeStruct((N, HIDP), f32),
        mesh=_sc_mesh,
        compiler_params=_sc_params,
        scratch_types=[
            pltpu.VMEM_SHARED((NPAD, Q), f32),
            pltpu.VMEM((SK, 128), i32), pltpu.VMEM((SCH, Q), f32),
            pltpu.VMEM((SK, 128), i32), pltpu.VMEM((SCH, Q), f32),
            pltpu.SemaphoreType.DMA,
            pltpu.SemaphoreType.DMA,
        ],
    )(e, ss, zeros)


def _run_object_mlp(x, agg, om_w1, om_b1, om_w2, om_b2):
    f32 = jnp.float32
    w1a = jnp.pad(om_w1[2:], ((0, HIDP - HID), (0, 0)))
    return pl.pallas_call(
        _object_mlp_body,
        grid=(N_GRID,),
        in_specs=[
            pl.BlockSpec((NTC, 2), lambda i: (i, 0)),
            pl.BlockSpec((NTC, HIDP), lambda i: (i, 0)),
            _full((2, HID)), _full((HIDP, HID)), _full((1, HID)),
            _full((HID, 2)), _full((1, 2)),
        ],
        out_specs=pl.BlockSpec((NTC, 2), lambda i: (i, 0)),
        out_shape=jax.ShapeDtypeStruct((N, 2), f32),
        compiler_params=pltpu.CompilerParams(
            dimension_semantics=("arbitrary",)),
    )(x, agg, om_w1[0:2], w1a, om_b1.reshape(1, HID),
      om_w2, om_b2.reshape(1, 2))


def _prep_indices(x, edge_index):
    sender = edge_index[0]
    receiver = edge_index[1]
    pad = EP - E
    sg = jnp.pad(sender, (0, pad)).reshape(ROWS, 128)
    rg = jnp.pad(receiver, (0, pad)).reshape(ROWS, 128)
    ss = jnp.pad(sender, (0, pad), constant_values=N).reshape(ROWS, 128)
    x8 = jnp.pad(x, ((0, 0), (0, 6)))
    zeros = jnp.zeros((NT, Q), jnp.float32)
    return x8, sg, rg, ss, zeros


def kernel(x, edge_index, rm_w1, rm_b1, rm_w2, rm_b2, rm_w3, rm_b3,
           rm_w4, rm_b4, om_w1, om_b1, om_w2, om_b2):
    x8, sg, rg, ss, zeros = _prep_indices(x, edge_index)
    spos, rpos = _run_gather(x8, sg, rg)
    e = _run_relation_mlp(spos, rpos, rm_w1, rm_b1, rm_w2, rm_b2,
                          rm_w3, rm_b3, rm_w4, rm_b4)
    agg = _run_scatter(e, ss, zeros)
    return _run_object_mlp(x, agg, om_w1, om_b1, om_w2, om_b2)
